# Initial kernel scaffold; baseline (speedup 1.0000x reference)
import jax
import jax.numpy as jnp
from jax.experimental import pallas as pl


def kernel(x, degree, W_in, b_in, W1, b1, W2, b2, ln_s, ln_b, out_ln_s, out_ln_b, W_out, b_out, edge_index):
    raise NotImplementedError("write your pallas kernel here")


# speedup vs baseline: 15.1068x; 15.1068x over previous
import functools

import jax
import jax.numpy as jnp
from jax import lax
from jax.experimental import pallas as pl
from jax.experimental.pallas import tpu as pltpu
from jax.experimental.pallas import tpu_sc as plsc

N = 10000
E = 320000
D_IN = 128
EMB = 128
HID = 128
OUT = 64
L = 3

NC = 2
NS = 16
NW = NC * NS
EPW = E // NW
CH = 80
NCHUNK = EPW // CH
RPB = 624
TAIL = N - NS * RPB

ROWS_B = 1000
GRID = N // ROWS_B


def _ln_rows(t, s, b):
    mu = jnp.mean(t, axis=-1, keepdims=True)
    var = jnp.mean((t - mu) ** 2, axis=-1, keepdims=True)
    return (t - mu) * lax.rsqrt(var + 1e-5) * s + b


def _in_body(x_ref, w_ref, b_ref, o_ref):
    o_ref[...] = jax.nn.relu(
        jnp.dot(x_ref[...], w_ref[...], preferred_element_type=jnp.float32)
        + b_ref[...]
    )


def _msg_body(h_ref, deg_ref, w_ref, b_ref, p_ref):
    inv = lax.rsqrt(jnp.maximum(deg_ref[...] + 1.0, 1.0))
    m = jax.nn.relu(
        jnp.dot(h_ref[...], w_ref[...], preferred_element_type=jnp.float32)
        + b_ref[...]
    )
    p_ref[...] = m * inv


def _upd_body(q0_ref, q1_ref, p_ref, h_ref, deg_ref, w_ref, b_ref, s_ref, lb_ref, o_ref):
    inv = lax.rsqrt(jnp.maximum(deg_ref[...] + 1.0, 1.0))
    agg = (q0_ref[...] + q1_ref[...] + p_ref[...]) * inv
    t = jnp.dot(agg, w_ref[...], preferred_element_type=jnp.float32) + b_ref[...]
    o_ref[...] = _ln_rows(t, s_ref[...], lb_ref[...]) + h_ref[...]


def _out_body(h_ref, s_ref, lb_ref, w_ref, b_ref, o_ref):
    t = _ln_rows(h_ref[...], s_ref[...], lb_ref[...])
    o_ref[...] = (
        jnp.dot(t, w_ref[...], preferred_element_type=jnp.float32) + b_ref[...]
    )


_in_call = pl.pallas_call(
    _in_body,
    grid=(GRID,),
    in_specs=[
        pl.BlockSpec((ROWS_B, D_IN), lambda i: (i, 0)),
        pl.BlockSpec((D_IN, EMB), lambda i: (0, 0)),
        pl.BlockSpec((1, EMB), lambda i: (0, 0)),
    ],
    out_specs=pl.BlockSpec((ROWS_B, EMB), lambda i: (i, 0)),
    out_shape=jax.ShapeDtypeStruct((N, EMB), jnp.float32),
)

_msg_call = pl.pallas_call(
    _msg_body,
    grid=(GRID,),
    in_specs=[
        pl.BlockSpec((ROWS_B, EMB), lambda i: (i, 0)),
        pl.BlockSpec((ROWS_B, 1), lambda i: (i, 0)),
        pl.BlockSpec((EMB, HID), lambda i: (0, 0)),
        pl.BlockSpec((1, HID), lambda i: (0, 0)),
    ],
    out_specs=pl.BlockSpec((ROWS_B, HID), lambda i: (i, 0)),
    out_shape=jax.ShapeDtypeStruct((N, HID), jnp.float32),
)

_upd_call = pl.pallas_call(
    _upd_body,
    grid=(GRID,),
    in_specs=[
        pl.BlockSpec((ROWS_B, HID), lambda i: (i, 0)),
        pl.BlockSpec((ROWS_B, HID), lambda i: (i, 0)),
        pl.BlockSpec((ROWS_B, HID), lambda i: (i, 0)),
        pl.BlockSpec((ROWS_B, EMB), lambda i: (i, 0)),
        pl.BlockSpec((ROWS_B, 1), lambda i: (i, 0)),
        pl.BlockSpec((HID, EMB), lambda i: (0, 0)),
        pl.BlockSpec((1, EMB), lambda i: (0, 0)),
        pl.BlockSpec((1, EMB), lambda i: (0, 0)),
        pl.BlockSpec((1, EMB), lambda i: (0, 0)),
    ],
    out_specs=pl.BlockSpec((ROWS_B, EMB), lambda i: (i, 0)),
    out_shape=jax.ShapeDtypeStruct((N, EMB), jnp.float32),
)

_out_call = pl.pallas_call(
    _out_body,
    grid=(GRID,),
    in_specs=[
        pl.BlockSpec((ROWS_B, EMB), lambda i: (i, 0)),
        pl.BlockSpec((1, EMB), lambda i: (0, 0)),
        pl.BlockSpec((1, EMB), lambda i: (0, 0)),
        pl.BlockSpec((EMB, OUT), lambda i: (0, 0)),
        pl.BlockSpec((1, OUT), lambda i: (0, 0)),
    ],
    out_specs=pl.BlockSpec((ROWS_B, OUT), lambda i: (i, 0)),
    out_shape=jax.ShapeDtypeStruct((N, OUT), jnp.float32),
)


def _sc_body(p_hbm, src_hbm, dst_hbm, zeros_hbm, out_hbm,
             src_v, dst_v, rows_v, acc, gsem, ssem):
    c = lax.axis_index("c")
    s = lax.axis_index("s")
    wid = c * NS + s
    pltpu.sync_copy(src_hbm.at[wid], src_v)
    pltpu.sync_copy(dst_hbm.at[wid], dst_v)
    pltpu.sync_copy(zeros_hbm.at[pl.ds(0, RPB)], acc.at[pl.ds(s * RPB, RPB)])

    @pl.when(s == 0)
    def _zero_tail():
        pltpu.sync_copy(zeros_hbm.at[pl.ds(0, TAIL)],
                        acc.at[pl.ds(NS * RPB, TAIL)])

    plsc.subcore_barrier()

    def chunk(i, carry):
        pltpu.async_copy(p_hbm.at[src_v.at[i]], rows_v, gsem).wait()
        pltpu.async_copy(rows_v, acc.at[dst_v.at[i]], ssem, add=True).wait()
        return carry

    lax.fori_loop(0, NCHUNK, chunk, 0)
    plsc.subcore_barrier()
    pltpu.sync_copy(acc.at[pl.ds(s * RPB, RPB)],
                    out_hbm.at[c].at[pl.ds(s * RPB, RPB)])

    @pl.when(s == 0)
    def _drain_tail():
        pltpu.sync_copy(acc.at[pl.ds(NS * RPB, TAIL)],
                        out_hbm.at[c].at[pl.ds(NS * RPB, TAIL)])


@functools.lru_cache(maxsize=None)
def _make_sc_call():
    return functools.partial(
        pl.kernel,
        out_type=jax.ShapeDtypeStruct((NC, N, EMB), jnp.float32),
        mesh=plsc.VectorSubcoreMesh(core_axis_name="c", subcore_axis_name="s",
                                    num_cores=NC, num_subcores=NS),
        scratch_types=[
            pltpu.VMEM((NCHUNK, CH), jnp.int32),
            pltpu.VMEM((NCHUNK, CH), jnp.int32),
            pltpu.VMEM((CH, EMB), jnp.float32),
            pltpu.VMEM_SHARED((N, EMB), jnp.float32),
            pltpu.SemaphoreType.DMA,
            pltpu.SemaphoreType.DMA,
        ],
    )(_sc_body)


def kernel(x, degree, W_in, b_in, W1, b1, W2, b2, ln_s, ln_b,
           out_ln_s, out_ln_b, W_out, b_out, edge_index):
    deg = degree.reshape(N, 1)
    src = edge_index[0].reshape(NW, NCHUNK, CH)
    dst = edge_index[1].reshape(NW, NCHUNK, CH)
    zeros = jnp.zeros((RPB, EMB), jnp.float32)

    h = _in_call(x, W_in, b_in.reshape(1, EMB))
    for l in range(L):
        p = _msg_call(h, deg, W1[l], b1[l].reshape(1, HID))
        q = _make_sc_call()(p, src, dst, zeros)
        h = _upd_call(q[0], q[1], p, h, deg, W2[l], b2[l].reshape(1, EMB),
                      ln_s[l].reshape(1, EMB), ln_b[l].reshape(1, EMB))
    return _out_call(h, out_ln_s.reshape(1, EMB), out_ln_b.reshape(1, EMB),
                     W_out, b_out.reshape(1, OUT))
